# baseline (speedup 1.0000x reference)
import functools

import jax
import jax.numpy as jnp
from jax import lax
from jax.experimental import pallas as pl
from jax.experimental.pallas import tpu as pltpu
from jax.experimental.pallas import tpu_sc as plsc

N = 10000
E = 320000
D = 128
HID = 2 * D
L = 5
G = 256
NUM_TASKS = 12
EPS = 1e-5

NC = 2
NS = 16
NW = NC * NS
EPW = E // NW
CH = 80
NCH = EPW // CH
NP = 10112
RPS = NP // NS

_f32 = jnp.float32
_i32 = jnp.int32

@functools.cache
def _get_sc_gather_scatter(tbl_rows):
    mesh = plsc.VectorSubcoreMesh(core_axis_name="c", subcore_axis_name="s")

    @functools.partial(
        pl.kernel,
        out_type=jax.ShapeDtypeStruct((NC, NP, D), _f32),
        mesh=mesh,
        scratch_types=[
            pltpu.VMEM((EPW,), _i32),
            pltpu.VMEM((NCH, CH), _i32),
            pltpu.VMEM((CH, D), _f32),
            pltpu.VMEM((CH, D), _f32),
            pltpu.VMEM_SHARED((NP, D), _f32),
            pltpu.SemaphoreType.DMA,
            pltpu.SemaphoreType.DMA,
        ],
    )
    def sc_gs(tbl_hbm, gidx_hbm, ridx_hbm, zeros_hbm, out_hbm,
              gidx, ridx, rows0, rows1, acc, sem0, sem1):
        c = lax.axis_index("c")
        s = lax.axis_index("s")
        w = s * NC + c
        pltpu.sync_copy(gidx_hbm.at[pl.ds(w * EPW, EPW)], gidx)
        pltpu.sync_copy(ridx_hbm.at[w], ridx)
        pltpu.sync_copy(zeros_hbm.at[pl.ds(s * RPS, RPS)],
                        acc.at[pl.ds(s * RPS, RPS)])
        plsc.subcore_barrier()

        bufs = ((rows0, sem0), (rows1, sem1))

        def gsl(ch):
            return gidx.at[pl.ds(ch * CH, CH)]

        pltpu.async_copy(tbl_hbm.at[gsl(0)], rows0, sem0)
        pltpu.async_copy(tbl_hbm.at[gsl(1)], rows1, sem1)

        def body(it, carry):
            g = it * 2
            for b in range(2):
                ch = g + b
                rows, sem = bufs[b]
                pltpu.make_async_copy(tbl_hbm.at[gsl(ch)], rows, sem).wait()
                pltpu.sync_copy(rows, acc.at[ridx.at[ch]], add=True)
                nxt = ch + 2

                @pl.when(nxt < NCH)
                def _():
                    pltpu.async_copy(tbl_hbm.at[gsl(nxt)], rows, sem)
            return carry

        lax.fori_loop(0, NCH // 2, body, 0)
        if NCH % 2:
            last = NCH - 1
            rows, sem = bufs[last % 2]
            pltpu.make_async_copy(tbl_hbm.at[gsl(last)], rows, sem).wait()
            pltpu.sync_copy(rows, acc.at[ridx.at[last]], add=True)
        plsc.subcore_barrier()
        pltpu.sync_copy(acc.at[pl.ds(s * RPS, RPS)],
                        out_hbm.at[c, pl.ds(s * RPS, RPS)])

    return sc_gs


def _sc_agg(h, col, row, zeros_nd):
    row3 = row.reshape(NW, NCH, CH)
    return _get_sc_gather_scatter(N)(h, col, row3, zeros_nd)


def _sc_counts(midx, row, eye256, zeros_nd):
    row3 = row.reshape(NW, NCH, CH)
    return _get_sc_gather_scatter(256)(eye256, midx, row3, zeros_nd)


def _embed_body(x_ref, t0_ref, a0_ref, a1_ref, h0_ref, midx_ref):
    xv = x_ref[...]
    x0 = xv[:, 0:1]
    x1 = xv[:, 1:2]
    iota = lax.broadcasted_iota(_i32, (N, 128), 1)
    cond = ((iota < 8) & (iota == x0)) | (
        (iota >= 8) & (iota < 16) & ((iota - 8) == x1))
    oh = jnp.where(cond, 1.0, 0.0).astype(_f32)
    h0_ref[...] = jnp.dot(oh, t0_ref[...], preferred_element_type=_f32, precision=lax.Precision.HIGHEST)
    lane = (lax.broadcasted_iota(_i32, (E // 128, 128), 1)
            + 128 * (lax.broadcasted_iota(_i32, (E // 128, 128), 0) % 2))
    midx_ref[...] = a0_ref[...] * 3 + a1_ref[...] + 16 * lane


_embed_call = pl.pallas_call(
    _embed_body,
    out_shape=(jax.ShapeDtypeStruct((N, D), _f32),
               jax.ShapeDtypeStruct((E // 128, 128), _i32)),
)


def _layer_body(aggp_ref, h_ref, cp_ref, etab_ref, w1_ref, b1_ref,
                w2_ref, b2_ref, g_ref, bb_ref, out_ref):
    cnt = cp_ref[0, :N] + cp_ref[1, :N]
    agg = (aggp_ref[0, :N] + aggp_ref[1, :N]) + jnp.dot(
        cnt, etab_ref[...], preferred_element_type=_f32,
        precision=lax.Precision.HIGHEST)
    xin = agg + h_ref[...]
    t = jnp.maximum(
        jnp.dot(xin, w1_ref[...], preferred_element_type=_f32) + b1_ref[...],
        0.0)
    y = jnp.dot(t, w2_ref[...], preferred_element_type=_f32) + b2_ref[...]
    mean = jnp.mean(y, axis=0, keepdims=True)
    yc = y - mean
    var = jnp.mean(yc * yc, axis=0, keepdims=True)
    out_ref[...] = jnp.maximum(
        g_ref[...] * yc / jnp.sqrt(var + EPS) + bb_ref[...], 0.0)


_layer_call = pl.pallas_call(
    _layer_body,
    out_shape=jax.ShapeDtypeStruct((N, D), _f32),
)


def _head_body(h_ref, batch_ref, fw_ref, fb_ref, w1_ref, b1_ref,
               w2_ref, b2_ref, w3_ref, b3_ref, out_ref):
    bidx = batch_ref[...]
    iota = lax.broadcasted_iota(_i32, (N, G), 1)
    oh = jnp.where(iota == bidx, 1.0, 0.0).astype(_f32)
    pooled = lax.dot_general(oh, h_ref[...], (((0,), (0,)), ((), ())),
                             preferred_element_type=_f32,
                             precision=lax.Precision.HIGHEST)
    z = jnp.maximum(
        jnp.dot(pooled, fw_ref[...], preferred_element_type=_f32)
        + fb_ref[...], 0.0)
    z = jnp.maximum(
        jnp.dot(z, w1_ref[...], preferred_element_type=_f32) + b1_ref[...],
        0.0)
    z = jnp.maximum(
        jnp.dot(z, w2_ref[...], preferred_element_type=_f32) + b2_ref[...],
        0.0)
    out_ref[...] = (jnp.dot(z, w3_ref[...], preferred_element_type=_f32)
                    + b3_ref[...])


_head_call = pl.pallas_call(
    _head_body,
    out_shape=jax.ShapeDtypeStruct((G, 128), _f32),
)


def kernel(x, edge_index, edge_attr, batch, x_emb1, x_emb2, e_emb1, e_emb2,
           mlp_W1, mlp_b1, mlp_W2, mlp_b2, bn_g, bn_b,
           feat_W, feat_b, p_W1, p_b1, p_W2, p_b2, p_W3, p_b3):
    x = x.astype(_i32)
    row = edge_index[0].astype(_i32)
    col = edge_index[1].astype(_i32)
    a0 = edge_attr[:, 0].astype(_i32)
    a1 = edge_attr[:, 1].astype(_i32)

    t0 = (jnp.zeros((128, D), _f32)
          .at[0:3].set(x_emb1[:3]).at[8:11].set(x_emb2))

    tab9 = (e_emb1[:, :3, None, :] + e_emb2[:, None, :, :]).reshape(L, 9, D)
    etab = jnp.concatenate([tab9, jnp.zeros((L, 119, D), _f32)], axis=1)

    eye256 = jnp.tile(jnp.eye(16, D, dtype=_f32), (256, 1))
    zeros_nd = jnp.zeros((NP, D), _f32)


    h, midx = _embed_call(x, t0, a0.reshape(E // 128, 128),
                          a1.reshape(E // 128, 128))
    cp = _sc_counts(midx.reshape(E), row, eye256, zeros_nd)

    for l in range(L):
        aggp = _sc_agg(h, col, row, zeros_nd)
        h = _layer_call(aggp, h, cp, etab[l],
                        mlp_W1[l], mlp_b1[l].reshape(1, HID),
                        mlp_W2[l], mlp_b2[l].reshape(1, D),
                        bn_g[l].reshape(1, D), bn_b[l].reshape(1, D))

    w3p = jnp.zeros((256, 128), _f32).at[:, :NUM_TASKS].set(p_W3)
    b3p = jnp.zeros((1, 128), _f32).at[:, :NUM_TASKS].set(p_b3.reshape(1, -1))
    out = _head_call(h, batch.astype(_i32).reshape(N, 1),
                     feat_W, feat_b.reshape(1, 512),
                     p_W1, p_b1.reshape(1, 256),
                     p_W2, p_b2.reshape(1, 256),
                     w3p, b3p)
    return out[:, :NUM_TASKS]


# speedup vs baseline: 1.0122x; 1.0122x over previous
import functools

import jax
import jax.numpy as jnp
from jax import lax
from jax.experimental import pallas as pl
from jax.experimental.pallas import tpu as pltpu
from jax.experimental.pallas import tpu_sc as plsc

N = 10000
E = 320000
D = 128
HID = 2 * D
L = 5
G = 256
NUM_TASKS = 12
EPS = 1e-5

NC = 2
NS = 16
NW = NC * NS
EPW = E // NW
CH = 80
NCH = EPW // CH
NP = 10112
RPS = NP // NS

_f32 = jnp.float32
_i32 = jnp.int32

@functools.cache
def _get_sc_gather_scatter(tbl_rows):
    mesh = plsc.VectorSubcoreMesh(core_axis_name="c", subcore_axis_name="s")

    @functools.partial(
        pl.kernel,
        out_type=jax.ShapeDtypeStruct((NC, NP, D), _f32),
        mesh=mesh,
        scratch_types=[
            pltpu.VMEM((EPW,), _i32),
            pltpu.VMEM((NCH, CH), _i32),
            pltpu.VMEM((CH, D), _f32),
            pltpu.VMEM((CH, D), _f32),
            pltpu.VMEM_SHARED((NP, D), _f32),
            pltpu.SemaphoreType.DMA,
            pltpu.SemaphoreType.DMA,
        ],
    )
    def sc_gs(tbl_hbm, gidx_hbm, ridx_hbm, zeros_hbm, out_hbm,
              gidx, ridx, rows0, rows1, acc, sem0, sem1):
        c = lax.axis_index("c")
        s = lax.axis_index("s")
        w = s * NC + c
        pltpu.sync_copy(gidx_hbm.at[pl.ds(w * EPW, EPW)], gidx)
        pltpu.sync_copy(ridx_hbm.at[w], ridx)
        pltpu.sync_copy(zeros_hbm.at[pl.ds(s * RPS, RPS)],
                        acc.at[pl.ds(s * RPS, RPS)])
        plsc.subcore_barrier()

        bufs = ((rows0, sem0), (rows1, sem1))

        def gsl(ch):
            return gidx.at[pl.ds(ch * CH, CH)]

        pltpu.async_copy(tbl_hbm.at[gsl(0)], rows0, sem0)
        pltpu.async_copy(tbl_hbm.at[gsl(1)], rows1, sem1)

        def body(it, carry):
            g = it * 2
            for b in range(2):
                ch = g + b
                rows, sem = bufs[b]
                pltpu.make_async_copy(tbl_hbm.at[gsl(ch)], rows, sem).wait()
                pltpu.sync_copy(rows, acc.at[ridx.at[ch]], add=True)
                nxt = ch + 2

                @pl.when(nxt < NCH)
                def _():
                    pltpu.async_copy(tbl_hbm.at[gsl(nxt)], rows, sem)
            return carry

        lax.fori_loop(0, NCH // 2, body, 0)
        if NCH % 2:
            last = NCH - 1
            rows, sem = bufs[last % 2]
            pltpu.make_async_copy(tbl_hbm.at[gsl(last)], rows, sem).wait()
            pltpu.sync_copy(rows, acc.at[ridx.at[last]], add=True)
        plsc.subcore_barrier()
        pltpu.sync_copy(acc.at[pl.ds(s * RPS, RPS)],
                        out_hbm.at[c, pl.ds(s * RPS, RPS)])

    return sc_gs


def _sc_agg(h, col, row, zeros_nd):
    row3 = row.reshape(NW, NCH, CH)
    return _get_sc_gather_scatter(N)(h, col, row3, zeros_nd)


def _sc_counts(midx, row, eye256, zeros_nd):
    row3 = row.reshape(NW, NCH, CH)
    return _get_sc_gather_scatter(256)(eye256, midx, row3, zeros_nd)


def _embed_body(x_ref, t0_ref, h0_ref):
    xv = x_ref[...]
    x0 = xv[:, 0:1]
    x1 = xv[:, 1:2]
    iota = lax.broadcasted_iota(_i32, (N, 128), 1)
    cond = ((iota < 8) & (iota == x0)) | (
        (iota >= 8) & (iota < 16) & ((iota - 8) == x1))
    oh = jnp.where(cond, 1.0, 0.0).astype(_f32)
    h0_ref[...] = jnp.dot(oh, t0_ref[...], preferred_element_type=_f32, precision=lax.Precision.HIGHEST)


_embed_call = pl.pallas_call(
    _embed_body,
    out_shape=jax.ShapeDtypeStruct((N, D), _f32),
)


def _midx_body(a0_ref, a1_ref, o_ref):
    lane = (lax.broadcasted_iota(_i32, (E // 128, 128), 1)
            + 128 * (lax.broadcasted_iota(_i32, (E // 128, 128), 0) % 2))
    o_ref[...] = a0_ref[...] * 3 + a1_ref[...] + 16 * lane


_midx_call = pl.pallas_call(
    _midx_body,
    out_shape=jax.ShapeDtypeStruct((E // 128, 128), _i32),
)


def _layer_body(aggp_ref, h_ref, cp_ref, etab_ref, w1_ref, b1_ref,
                w2_ref, b2_ref, g_ref, bb_ref, out_ref):
    cnt = cp_ref[0, :N] + cp_ref[1, :N]
    agg = (aggp_ref[0, :N] + aggp_ref[1, :N]) + jnp.dot(
        cnt, etab_ref[...], preferred_element_type=_f32,
        precision=lax.Precision.HIGHEST)
    xin = agg + h_ref[...]
    t = jnp.maximum(
        jnp.dot(xin, w1_ref[...], preferred_element_type=_f32) + b1_ref[...],
        0.0)
    y = jnp.dot(t, w2_ref[...], preferred_element_type=_f32) + b2_ref[...]
    mean = jnp.mean(y, axis=0, keepdims=True)
    yc = y - mean
    var = jnp.mean(yc * yc, axis=0, keepdims=True)
    out_ref[...] = jnp.maximum(
        g_ref[...] * yc / jnp.sqrt(var + EPS) + bb_ref[...], 0.0)


_layer_call = pl.pallas_call(
    _layer_body,
    out_shape=jax.ShapeDtypeStruct((N, D), _f32),
)


def _head_body(h_ref, batch_ref, fw_ref, fb_ref, w1_ref, b1_ref,
               w2_ref, b2_ref, w3_ref, b3_ref, out_ref):
    bidx = batch_ref[...]
    iota = lax.broadcasted_iota(_i32, (N, G), 1)
    oh = jnp.where(iota == bidx, 1.0, 0.0).astype(_f32)
    pooled = lax.dot_general(oh, h_ref[...], (((0,), (0,)), ((), ())),
                             preferred_element_type=_f32,
                             precision=lax.Precision.HIGHEST)
    z = jnp.maximum(
        jnp.dot(pooled, fw_ref[...], preferred_element_type=_f32)
        + fb_ref[...], 0.0)
    z = jnp.maximum(
        jnp.dot(z, w1_ref[...], preferred_element_type=_f32) + b1_ref[...],
        0.0)
    z = jnp.maximum(
        jnp.dot(z, w2_ref[...], preferred_element_type=_f32) + b2_ref[...],
        0.0)
    out_ref[...] = (jnp.dot(z, w3_ref[...], preferred_element_type=_f32)
                    + b3_ref[...])


_head_call = pl.pallas_call(
    _head_body,
    out_shape=jax.ShapeDtypeStruct((G, 128), _f32),
)


def kernel(x, edge_index, edge_attr, batch, x_emb1, x_emb2, e_emb1, e_emb2,
           mlp_W1, mlp_b1, mlp_W2, mlp_b2, bn_g, bn_b,
           feat_W, feat_b, p_W1, p_b1, p_W2, p_b2, p_W3, p_b3):
    x = x.astype(_i32)
    row = edge_index[0].astype(_i32)
    col = edge_index[1].astype(_i32)
    a0 = edge_attr[:, 0].astype(_i32)
    a1 = edge_attr[:, 1].astype(_i32)

    t0 = (jnp.zeros((128, D), _f32)
          .at[0:3].set(x_emb1[:3]).at[8:11].set(x_emb2))

    tab9 = (e_emb1[:, :3, None, :] + e_emb2[:, None, :, :]).reshape(L, 9, D)
    etab = jnp.concatenate([tab9, jnp.zeros((L, 119, D), _f32)], axis=1)

    eye256 = jnp.tile(jnp.eye(16, D, dtype=_f32), (256, 1))
    zeros_nd = jnp.zeros((NP, D), _f32)


    h = _embed_call(x, t0)
    midx = _midx_call(a0.reshape(E // 128, 128),
                      a1.reshape(E // 128, 128)).reshape(E)
    cp = _sc_counts(midx, row, eye256, zeros_nd)

    for l in range(L):
        aggp = _sc_agg(h, col, row, zeros_nd)
        h = _layer_call(aggp, h, cp, etab[l],
                        mlp_W1[l], mlp_b1[l].reshape(1, HID),
                        mlp_W2[l], mlp_b2[l].reshape(1, D),
                        bn_g[l].reshape(1, D), bn_b[l].reshape(1, D))

    w3p = jnp.zeros((256, 128), _f32).at[:, :NUM_TASKS].set(p_W3)
    b3p = jnp.zeros((1, 128), _f32).at[:, :NUM_TASKS].set(p_b3.reshape(1, -1))
    out = _head_call(h, batch.astype(_i32).reshape(N, 1),
                     feat_W, feat_b.reshape(1, 512),
                     p_W1, p_b1.reshape(1, 256),
                     p_W2, p_b2.reshape(1, 256),
                     w3p, b3p)
    return out[:, :NUM_TASKS]
